# Initial kernel scaffold; baseline (speedup 1.0000x reference)
import jax
import jax.numpy as jnp
from jax.experimental import pallas as pl


def kernel(H, idx, X_e, W1, W2):
    raise NotImplementedError("write your pallas kernel here")


# speedup vs baseline: 2.7524x; 2.7524x over previous
import functools

import jax
import jax.numpy as jnp
from jax import lax
from jax.experimental import pallas as pl
from jax.experimental.pallas import tpu as pltpu
from jax.experimental.pallas import tpu_sc as plsc

N_NODES = 10000
N_EDGES = 320000
D = 128

NC, NS, L = 2, 16, 16
NW = NC * NS
EPW = N_EDGES // NW
CHUNK = 80
NCHUNK = EPW // CHUNK
N_PAD = 10240
RPT = N_PAD // NS
ZROWS = 128


def _edge_sc(g_hbm, src_hbm, dst_hbm, ex_hbm, out_hbm,
             src_v, dst_v, g_v, ex_v, zb_v, agg_sh, sem):
    cid = lax.axis_index("c")
    sid = lax.axis_index("s")
    wid = sid * NC + cid

    def zrow(r, carry):
        for j in range(D // L):
            zb_v[r, pl.ds(j * L, L)] = jnp.zeros((L,), jnp.float32)
        return carry

    lax.fori_loop(0, ZROWS, zrow, 0)
    for k in range(RPT // ZROWS):
        pltpu.sync_copy(zb_v, agg_sh.at[pl.ds(sid * RPT + k * ZROWS, ZROWS)])
    plsc.subcore_barrier()

    base = wid * EPW

    def chunk(c, carry):
        eb = pl.multiple_of(base + c * CHUNK, 8)
        pltpu.sync_copy(src_hbm.at[pl.ds(eb, CHUNK)], src_v)
        pltpu.sync_copy(dst_hbm.at[pl.ds(eb, CHUNK)], dst_v)
        pltpu.async_copy(g_hbm.at[src_v], g_v, sem).wait()
        pltpu.sync_copy(ex_hbm.at[pl.ds(eb, CHUNK)], ex_v)

        def row(r, rcarry):
            for j in range(D // L):
                s = pl.ds(j * L, L)
                g_v[r, s] = jnp.maximum(g_v[r, s] + ex_v[r, s], 0.0)
            return rcarry

        lax.fori_loop(0, CHUNK, row, 0)
        pltpu.sync_copy(g_v, agg_sh.at[dst_v], add=True)
        return carry

    lax.fori_loop(0, NCHUNK, chunk, 0)
    plsc.subcore_barrier()

    pltpu.sync_copy(agg_sh.at[pl.ds(sid * RPT, RPT)],
                    out_hbm.at[cid, pl.ds(sid * RPT, RPT)])


def _mm_body(x_ref, w_ref, o_ref):
    o_ref[...] = jnp.dot(x_ref[...], w_ref[...],
                         preferred_element_type=jnp.float32)


def _final_body(h_ref, p_ref, w2h_ref, w2a_ref, o_ref):
    agg = p_ref[0, :N_NODES] + p_ref[1, :N_NODES]
    y = (jnp.dot(h_ref[...], w2h_ref[...], preferred_element_type=jnp.float32)
         + jnp.dot(agg, w2a_ref[...], preferred_element_type=jnp.float32))
    o_ref[...] = jnp.maximum(y, 0.0) + h_ref[...]


@jax.jit
def kernel(H, idx, X_e, W1, W2):
    idx = idx.astype(jnp.int32)
    src, dst = idx[0], idx[1]

    G = pl.pallas_call(
        _mm_body,
        out_shape=jax.ShapeDtypeStruct((N_NODES, D), jnp.float32),
    )(H, W1[:D])

    n_eb = 32
    Ex = pl.pallas_call(
        _mm_body,
        grid=(n_eb,),
        in_specs=[
            pl.BlockSpec((N_EDGES // n_eb, 16), lambda i: (i, 0)),
            pl.BlockSpec((16, D), lambda i: (0, 0)),
        ],
        out_specs=pl.BlockSpec((N_EDGES // n_eb, D), lambda i: (i, 0)),
        out_shape=jax.ShapeDtypeStruct((N_EDGES, D), jnp.float32),
    )(X_e, W1[D:])

    mesh = plsc.VectorSubcoreMesh(core_axis_name="c", subcore_axis_name="s",
                                  num_cores=NC, num_subcores=NS)
    partials = pl.kernel(
        _edge_sc,
        out_type=jax.ShapeDtypeStruct((NC, N_PAD, D), jnp.float32),
        mesh=mesh,
        scratch_types=[
            pltpu.VMEM((CHUNK,), jnp.int32),
            pltpu.VMEM((CHUNK,), jnp.int32),
            pltpu.VMEM((CHUNK, D), jnp.float32),
            pltpu.VMEM((CHUNK, D), jnp.float32),
            pltpu.VMEM((ZROWS, D), jnp.float32),
            pltpu.VMEM_SHARED((N_PAD, D), jnp.float32),
            pltpu.SemaphoreType.DMA,
        ],
    )(G, src, dst, Ex)

    out = pl.pallas_call(
        _final_body,
        out_shape=jax.ShapeDtypeStruct((N_NODES, D), jnp.float32),
    )(H, partials, W2[:D], W2[D:])
    return out


# speedup vs baseline: 4.2698x; 1.5513x over previous
import jax
import jax.numpy as jnp
from jax import lax
from jax.experimental import pallas as pl
from jax.experimental.pallas import tpu as pltpu
from jax.experimental.pallas import tpu_sc as plsc

N_NODES = 10000
N_EDGES = 320000
D = 128

NC, NS, L = 2, 16, 16
NW = NC * NS
EPW = N_EDGES // NW
CHUNK = 40
NCHUNK = EPW // CHUNK
RPT = 640


def _edge_sc(g_hbm, src_hbm, dst_hbm, ex_hbm, out_hbm,
             srcs_v, d0, d1, g0, g1, e0, e1, agg_sh,
             sg0, sg1, sd0, sd1, se0, se1, ss0, ss1):
    cid = lax.axis_index("c")
    sid = lax.axis_index("s")
    wid = sid * NC + cid
    base = wid * EPW

    pltpu.sync_copy(src_hbm.at[pl.ds(base, EPW)], srcs_v)

    def zrow(r, carry):
        for j in range(D // L):
            e0[r, pl.ds(j * L, L)] = jnp.zeros((L,), jnp.float32)
        return carry

    lax.fori_loop(0, CHUNK, zrow, 0)
    n_zero = jnp.where(sid == NS - 1, (N_NODES - (NS - 1) * RPT) // CHUNK,
                       RPT // CHUNK)

    def zcopy(k, carry):
        pltpu.sync_copy(e0, agg_sh.at[pl.ds(sid * RPT + k * CHUNK, CHUNK)])
        return carry

    lax.fori_loop(0, n_zero, zcopy, 0)

    def fetch(c, gbuf, dbuf, ebuf, gsem, dsem, esem):
        pltpu.async_copy(g_hbm.at[srcs_v.at[pl.ds(c * CHUNK, CHUNK)]],
                         gbuf, gsem)
        pltpu.async_copy(dst_hbm.at[pl.ds(base + c * CHUNK, CHUNK)],
                         dbuf, dsem)
        pltpu.async_copy(
            ex_hbm.at[pl.ds(base + c * CHUNK, CHUNK)], ebuf, esem)

    def wait_fetch(c, gbuf, dbuf, ebuf, gsem, dsem, esem):
        pltpu.make_async_copy(g_hbm.at[srcs_v.at[pl.ds(c * CHUNK, CHUNK)]],
                              gbuf, gsem).wait()
        pltpu.make_async_copy(dst_hbm.at[pl.ds(base + c * CHUNK, CHUNK)],
                              dbuf, dsem).wait()
        pltpu.make_async_copy(
            ex_hbm.at[pl.ds(base + c * CHUNK, CHUNK)], ebuf, esem).wait()

    def compute(gbuf, ebuf):
        def row(r, carry):
            for j in range(D // L):
                s = pl.ds(j * L, L)
                gbuf[r, s] = jnp.maximum(gbuf[r, s] + ebuf[r, s], 0.0)
            return carry

        lax.fori_loop(0, CHUNK, row, 0)

    def scatter(dbuf, gbuf, ssem):
        pltpu.async_copy(gbuf, agg_sh.at[dbuf], ssem, add=True)

    def wait_scatter(dbuf, gbuf, ssem):
        pltpu.make_async_copy(gbuf, agg_sh.at[dbuf], ssem).wait()

    fetch(0, g0, d0, e0, sg0, sd0, se0)
    plsc.subcore_barrier()

    def step(g, carry):
        c0 = 2 * g
        c1 = c0 + 1
        wait_fetch(c0, g0, d0, e0, sg0, sd0, se0)

        @pl.when(g > 0)
        def _():
            wait_scatter(d1, g1, ss1)

        fetch(c1, g1, d1, e1, sg1, sd1, se1)
        compute(g0, e0)
        scatter(d0, g0, ss0)

        wait_fetch(c1, g1, d1, e1, sg1, sd1, se1)
        wait_scatter(d0, g0, ss0)

        @pl.when(g < NCHUNK // 2 - 1)
        def _():
            fetch(c0 + 2, g0, d0, e0, sg0, sd0, se0)

        compute(g1, e1)
        scatter(d1, g1, ss1)
        return carry

    lax.fori_loop(0, NCHUNK // 2, step, 0)
    wait_scatter(d1, g1, ss1)
    plsc.subcore_barrier()

    n_out = jnp.where(sid == NS - 1, (N_NODES - (NS - 1) * RPT) // 80,
                      RPT // 80)

    def wcopy(k, carry):
        s = pl.ds(sid * RPT + k * 80, 80)
        pltpu.sync_copy(agg_sh.at[s], out_hbm.at[cid, s])
        return carry

    lax.fori_loop(0, n_out, wcopy, 0)


def _mm_body(x_ref, w_ref, o_ref):
    o_ref[...] = jnp.dot(x_ref[...], w_ref[...],
                         preferred_element_type=jnp.float32)


def _final_body(h_ref, p_ref, w2h_ref, w2a_ref, o_ref):
    agg = p_ref[0] + p_ref[1]
    y = (jnp.dot(h_ref[...], w2h_ref[...], preferred_element_type=jnp.float32)
         + jnp.dot(agg, w2a_ref[...], preferred_element_type=jnp.float32))
    o_ref[...] = jnp.maximum(y, 0.0) + h_ref[...]


@jax.jit
def kernel(H, idx, X_e, W1, W2):
    idx = idx.astype(jnp.int32)
    src, dst = idx[0], idx[1]

    G = pl.pallas_call(
        _mm_body,
        out_shape=jax.ShapeDtypeStruct((N_NODES, D), jnp.float32),
    )(H, W1[:D])

    n_eb = 32
    Ex = pl.pallas_call(
        _mm_body,
        grid=(n_eb,),
        in_specs=[
            pl.BlockSpec((N_EDGES // n_eb, 16), lambda i: (i, 0)),
            pl.BlockSpec((16, D), lambda i: (0, 0)),
        ],
        out_specs=pl.BlockSpec((N_EDGES // n_eb, D), lambda i: (i, 0)),
        out_shape=jax.ShapeDtypeStruct((N_EDGES, D), jnp.float32),
    )(X_e, W1[D:])

    mesh = plsc.VectorSubcoreMesh(core_axis_name="c", subcore_axis_name="s",
                                  num_cores=NC, num_subcores=NS)
    partials = pl.kernel(
        _edge_sc,
        out_type=jax.ShapeDtypeStruct((NC, N_NODES, D), jnp.float32),
        mesh=mesh,
        scratch_types=[
            pltpu.VMEM((EPW,), jnp.int32),
            pltpu.VMEM((CHUNK,), jnp.int32),
            pltpu.VMEM((CHUNK,), jnp.int32),
            pltpu.VMEM((CHUNK, D), jnp.float32),
            pltpu.VMEM((CHUNK, D), jnp.float32),
            pltpu.VMEM((CHUNK, D), jnp.float32),
            pltpu.VMEM((CHUNK, D), jnp.float32),
            pltpu.VMEM_SHARED((N_NODES, D), jnp.float32),
            pltpu.SemaphoreType.DMA,
            pltpu.SemaphoreType.DMA,
            pltpu.SemaphoreType.DMA,
            pltpu.SemaphoreType.DMA,
            pltpu.SemaphoreType.DMA,
            pltpu.SemaphoreType.DMA,
            pltpu.SemaphoreType.DMA,
            pltpu.SemaphoreType.DMA,
        ],
    )(G, src, dst, Ex)

    out = pl.pallas_call(
        _final_body,
        out_shape=jax.ShapeDtypeStruct((N_NODES, D), jnp.float32),
    )(H, partials, W2[:D], W2[D:])
    return out
